# Initial kernel scaffold; baseline (speedup 1.0000x reference)
import jax
import jax.numpy as jnp
from jax.experimental import pallas as pl


def kernel(x, edge_index, closenes_feature, batch, W1, b1, W2, b2, W3, b3, Wc, bc, Wa1, ba1, Wa2, ba2):
    raise NotImplementedError("write your pallas kernel here")


# speedup vs baseline: 14.3610x; 14.3610x over previous
import functools

import jax
import jax.numpy as jnp
from jax import lax
from jax.experimental import pallas as pl
from jax.experimental.pallas import tpu as pltpu
from jax.experimental.pallas import tpu_sc as plsc

N = 10000
E = 320000
D = 128
H = 64
G = 64
CF = 5

NC = 2
NS = 16
NW = NC * NS
LANES = 128
N_PAD = 10240
RPT = N_PAD // NS
CH = 79
EPT = CH * LANES
E_PAD = EPT * NW

_f32 = jnp.float32
_i32 = jnp.int32

_sc_mesh = plsc.VectorSubcoreMesh(core_axis_name="c", subcore_axis_name="s")


@functools.partial(
    pl.kernel,
    out_type=jax.ShapeDtypeStruct((NC, N_PAD), _f32),
    mesh=_sc_mesh,
    scratch_types=[
        pltpu.VMEM((CH, LANES), _i32),
        pltpu.VMEM((RPT,), _f32),
        pltpu.VMEM_SHARED((N_PAD,), _f32),
        pltpu.SemaphoreType.DMA,
    ],
)
def _deg_kernel(dst_hbm, out_hbm, dstv, buf, sdeg, sem):
    cid = lax.axis_index("c")
    sid = lax.axis_index("s")
    wid = cid * NS + sid

    zeros16 = jnp.zeros((16,), _f32)

    def zero_body(i, _):
        buf[pl.ds(i * 16, 16)] = zeros16
        return 0

    lax.fori_loop(0, RPT // 16, zero_body, 0)
    pltpu.sync_copy(buf, sdeg.at[pl.ds(sid * RPT, RPT)])

    pltpu.sync_copy(dst_hbm.at[wid], dstv)

    ones16 = jnp.ones((16,), _f32)

    def ones_body(i, _):
        buf[pl.ds(i * 16, 16)] = ones16
        return 0

    lax.fori_loop(0, LANES // 16, ones_body, 0)
    plsc.subcore_barrier()

    def chunk(j, _):
        pltpu.async_copy(buf.at[pl.ds(0, LANES)], sdeg.at[dstv.at[j]],
                         sem, add=True).wait()
        return 0

    lax.fori_loop(0, CH, chunk, 0)

    plsc.subcore_barrier()
    pltpu.sync_copy(sdeg.at[pl.ds(sid * RPT, RPT)],
                    out_hbm.at[cid, pl.ds(sid * RPT, RPT)])


@functools.partial(
    pl.kernel,
    out_type=jax.ShapeDtypeStruct((NC, N_PAD, H), _f32),
    mesh=_sc_mesh,
    scratch_types=[
        pltpu.VMEM((CH, LANES), _i32),
        pltpu.VMEM((CH, LANES), _i32),
        pltpu.VMEM((LANES, H), _f32),
        pltpu.VMEM_SHARED((N_PAD, H), _f32),
        pltpu.SemaphoreType.DMA,
        pltpu.SemaphoreType.DMA,
    ],
    compiler_params=pltpu.CompilerParams(use_tc_tiling_on_sc=False),
)
def _mp_kernel(z_hbm, src_hbm, dst_hbm, out_hbm, srcv, dstv, rows, acc, gsem, ssem):
    cid = lax.axis_index("c")
    sid = lax.axis_index("s")
    wid = cid * NS + sid

    zeros16 = jnp.zeros((16,), _f32)

    def zrow(i, _):
        for l in range(H // 16):
            rows[i, pl.ds(l * 16, 16)] = zeros16
        return 0

    lax.fori_loop(0, LANES, zrow, 0)
    for b in range(RPT // LANES):
        pltpu.sync_copy(rows, acc.at[pl.ds(sid * RPT + b * LANES, LANES)])

    pltpu.sync_copy(src_hbm.at[wid], srcv)
    pltpu.sync_copy(dst_hbm.at[wid], dstv)
    plsc.subcore_barrier()

    def chunk(j, _):
        pltpu.async_copy(z_hbm.at[srcv.at[j]], rows, gsem).wait()
        pltpu.async_copy(rows, acc.at[dstv.at[j]], ssem, add=True).wait()
        return 0

    lax.fori_loop(0, CH, chunk, 0)

    plsc.subcore_barrier()
    pltpu.sync_copy(acc.at[pl.ds(sid * RPT, RPT)],
                    out_hbm.at[cid, pl.ds(sid * RPT, RPT)])


def _k1_body(dega, degb, x, w1, z1, dinv):
    deg = 1.0 + dega[...] + degb[...]
    di = lax.rsqrt(deg)
    dinv[...] = di
    z1[...] = jnp.dot(x[...], w1[...], preferred_element_type=_f32) * di


def _k2_body(ya, yb, z, dinv, b, w, zn):
    di = dinv[...]
    h = jnp.maximum(di * (ya[...] + yb[...] + z[...]) + b[...], 0.0)
    zn[...] = jnp.dot(h, w[...], preferred_element_type=_f32) * di


def _k3_body(ya, yb, z, dinv, b, h_out):
    h_out[...] = jnp.maximum(
        dinv[...] * (ya[...] + yb[...] + z[...]) + b[...], 0.0)


def _pool_body(h3, b64, clos, wc, bc, wa1, ba1, wa2, ba2, out, att_scr):
    h = h3[...]
    bb = b64[...]
    gid = lax.broadcasted_iota(_i32, (N, G), 1)
    mbool = bb == gid
    mt = mbool.astype(_f32)

    wc64 = wc[...] * jnp.ones((1, G), _f32)
    c_b = jnp.dot(clos[...], wc64, preferred_element_type=_f32) + bc[0, 0]

    eye = (lax.broadcasted_iota(_i32, (G, G), 0)
           == lax.broadcasted_iota(_i32, (G, G), 1)).astype(_f32)
    ones_g = jnp.ones((G, G), _f32)

    neg_inf = _f32(-jnp.inf)
    m = jnp.max(jnp.where(mbool, c_b, neg_inf), axis=0, keepdims=True)
    mg = jnp.where(m == neg_inf, 0.0, m)
    m_bc = jnp.dot(eye * mg, ones_g, preferred_element_type=_f32)
    mnode = jnp.dot(mt, m_bc, preferred_element_type=_f32)
    e_b = jnp.exp(c_b - mnode)
    s = jnp.sum(mt * e_b, axis=0, keepdims=True)
    counts = jnp.sum(mt, axis=0, keepdims=True)
    factor = jnp.where(counts > 0, counts / s, 0.0)
    f_bc = jnp.dot(eye * factor, ones_g, preferred_element_type=_f32)
    fnode = jnp.dot(mt, f_bc, preferred_element_type=_f32)
    wh = e_b * fnode * h

    def seg_body(g, _):
        sel = jnp.where(bb == g, wh, neg_inf)
        att_scr[pl.ds(g, 1), :] = jnp.max(sel, axis=0, keepdims=True)
        return 0

    lax.fori_loop(0, G, seg_body, 0)

    att = att_scr[...]
    att = jnp.where(att == neg_inf, 0.0, att)
    a1 = jnp.maximum(jnp.dot(att, wa1[...], preferred_element_type=_f32)
                     + ba1[...], 0.0)
    out[...] = jnp.dot(a1, wa2[...], preferred_element_type=_f32) + ba2[...]


def _tc_call(body, out_shapes, *args, scratch_shapes=()):
    return pl.pallas_call(
        body,
        out_shape=out_shapes,
        scratch_shapes=scratch_shapes,
    )(*args)


def kernel(x, edge_index, closenes_feature, batch, W1, b1, W2, b2, W3, b3,
           Wc, bc, Wa1, ba1, Wa2, ba2):
    src = edge_index[0]
    dst = edge_index[1]
    pad = jnp.full((E_PAD - E,), N, _i32)
    src3 = jnp.concatenate([src, pad]).reshape(NW, CH, LANES)
    dst3 = jnp.concatenate([dst, pad]).reshape(NW, CH, LANES)
    x_pad = jnp.pad(x, ((0, N_PAD - N), (0, 0)))

    degp = _deg_kernel(dst3)
    dega = degp[0].reshape(N_PAD, 1)
    degb = degp[1].reshape(N_PAD, 1)

    z1, dinv = _tc_call(
        _k1_body,
        (jax.ShapeDtypeStruct((N_PAD, H), _f32),
         jax.ShapeDtypeStruct((N_PAD, 1), _f32)),
        dega, degb, x_pad, W1)

    y1 = _mp_kernel(z1, src3, dst3)
    z2 = _tc_call(_k2_body, jax.ShapeDtypeStruct((N_PAD, H), _f32),
                  y1[0], y1[1], z1, dinv, b1.reshape(1, H), W2)
    y2 = _mp_kernel(z2, src3, dst3)
    z3 = _tc_call(_k2_body, jax.ShapeDtypeStruct((N_PAD, H), _f32),
                  y2[0], y2[1], z2, dinv, b2.reshape(1, H), W3)
    y3 = _mp_kernel(z3, src3, dst3)

    h3 = _tc_call(_k3_body, jax.ShapeDtypeStruct((N_PAD, H), _f32),
                  y3[0], y3[1], z3, dinv, b3.reshape(1, H))
    b64 = jnp.broadcast_to(batch.reshape(N, 1), (N, G))
    out = _tc_call(
        _pool_body,
        jax.ShapeDtypeStruct((G, 1), _f32),
        h3[:N], b64, closenes_feature,
        Wc, bc.reshape(1, 1), Wa1, ba1.reshape(1, 16),
        Wa2, ba2.reshape(1, 1),
        scratch_shapes=[pltpu.VMEM((G, H), _f32)],
    )
    return out
